# baseline (speedup 1.0000x reference)
import functools

import jax
import jax.numpy as jnp
from jax.experimental import pallas as pl
from jax.experimental.pallas import tpu as pltpu

_NEG = -1e30


def _loss_kernel(hs_ref, tids_ref, Wr_ref, br_ref, cw_ref, cb_ref, out_ref,
                 wcat_ref, *, n_total, r):
    i = pl.program_id(0)

    @pl.when(i == 0)
    def _build_wcat():
        for j in range(cw_ref.shape[0] // 2):
            blk = jnp.concatenate(
                [cw_ref[2 * j], cw_ref[2 * j + 1]], axis=-1)
            wcat_ref[:, 2 * j * r:(2 * j + 2) * r] = blk.astype(jnp.bfloat16)

    tids = tids_ref[...]
    rows = tids // r
    cols = tids - rows * r
    hs = hs_ref[...].astype(jnp.bfloat16)

    p = jnp.dot(hs, wcat_ref[...], preferred_element_type=jnp.float32)
    p = p + cb_ref[...]

    lane = jax.lax.broadcasted_iota(jnp.int32, p.shape, 1)
    in_slice = (lane // r) == rows
    masked = jnp.where(in_slice, p, _NEG)
    m = jnp.max(masked, axis=-1, keepdims=True)
    s = jnp.sum(jnp.exp(masked - m), axis=-1, keepdims=True)
    lse = m + jnp.log(s)
    sel = jnp.sum(jnp.where(lane == tids, p, 0.0), axis=-1, keepdims=True)
    nll_col = jnp.sum(lse - sel, axis=0, keepdims=True)

    q = jnp.dot(hs, Wr_ref[...].astype(jnp.bfloat16),
                preferred_element_type=jnp.float32)
    q = q + br_ref[...]
    lane_r = jax.lax.broadcasted_iota(jnp.int32, q.shape, 1)
    mq = jnp.max(q, axis=-1, keepdims=True)
    sq = jnp.sum(jnp.exp(q - mq), axis=-1, keepdims=True)
    lse_q = mq + jnp.log(sq)
    sel_q = jnp.sum(jnp.where(lane_r == rows, q, 0.0), axis=-1, keepdims=True)
    nll_row = jnp.sum(lse_q - sel_q, axis=0, keepdims=True)

    partial = (nll_col + nll_row) / n_total

    @pl.when(i == 0)
    def _init():
        out_ref[...] = jnp.zeros_like(out_ref)

    out_ref[...] += partial


@jax.jit
def kernel(hidden_states, target_ids, Wr, br, col_weight, col_bias):
    d = hidden_states.shape[-1]
    r = br.shape[0]
    hs = hidden_states.reshape(-1, d)
    n = hs.shape[0]
    tids = target_ids.reshape(n, 1).astype(jnp.int32)
    cb_flat = col_bias.reshape(1, r * r)

    tb = 512
    grid = (n // tb,)

    out = pl.pallas_call(
        functools.partial(_loss_kernel, n_total=n, r=r),
        grid=grid,
        in_specs=[
            pl.BlockSpec((tb, d), lambda i: (i, 0)),
            pl.BlockSpec((tb, 1), lambda i: (i, 0)),
            pl.BlockSpec((d, r), lambda i: (0, 0)),
            pl.BlockSpec((1, r), lambda i: (0, 0)),
            pl.BlockSpec((r, d, r), lambda i: (0, 0, 0)),
            pl.BlockSpec((1, r * r), lambda i: (0, 0)),
        ],
        out_specs=pl.BlockSpec((1, 1), lambda i: (0, 0)),
        out_shape=jax.ShapeDtypeStruct((1, 1), jnp.float32),
        scratch_shapes=[pltpu.VMEM((d, r * r), jnp.bfloat16)],
        compiler_params=pltpu.CompilerParams(
            dimension_semantics=("arbitrary",)),
    )(hs, tids, Wr, br.reshape(1, r), col_weight, cb_flat)
    return out[0, 0]
